# Initial kernel scaffold; baseline (speedup 1.0000x reference)
import jax
import jax.numpy as jnp
from jax.experimental import pallas as pl


def kernel(node_ids, edge_index, edge_type, ent_embeds, rel_embeds, W_msg1, W_self1, W_rel1, W_msg2, W_self2):
    raise NotImplementedError("write your pallas kernel here")


# speedup vs baseline: 3.2784x; 3.2784x over previous
import functools

import jax
import jax.numpy as jnp
from jax import lax
from jax.experimental import pallas as pl
from jax.experimental.pallas import tpu as pltpu
from jax.experimental.pallas import tpu_sc as plsc

N = 10000
E = 320000
D = 128
OUT = 64
R = 256

NC = 2
NS = 16
NW = NC * NS

NP = 10240
ROWS_PER_TILE = NP // NW
ACC_ROWS = NP // NS
H_GROUP = 80
G = 128
GROUPS = 79
EPT = G * GROUPS
EP = EPT * NW
ACC_ZERO_BLKS = ACC_ROWS // G

_mesh = plsc.VectorSubcoreMesh(core_axis_name="c", subcore_axis_name="s")


def _zero_vmem_2d(ref, nrows, ncols):
    z = jnp.zeros((16,), jnp.float32)

    def body(i, _):
        r = i // (ncols // 16)
        c = (i % (ncols // 16)) * 16
        ref[r, pl.ds(c, 16)] = z
        return 0

    lax.fori_loop(0, nrows * (ncols // 16), body, 0)


def _fill_vmem_1d(ref, n, val):
    v = jnp.full((16,), val, jnp.float32)

    def body(i, _):
        ref[pl.ds(i * 16, 16)] = v
        return 0

    lax.fori_loop(0, n // 16, body, 0)


@functools.partial(
    pl.kernel,
    out_type=[
        jax.ShapeDtypeStruct((NP, D), jnp.float32),
        jax.ShapeDtypeStruct((NC, NP, D), jnp.float32),
        jax.ShapeDtypeStruct((NC * NP,), jnp.float32),
    ],
    mesh=_mesh,
    scratch_types=[
        pltpu.VMEM((G,), jnp.int32),
        pltpu.VMEM((G,), jnp.int32),
        pltpu.VMEM((G,), jnp.int32),
        pltpu.VMEM((G,), jnp.int32),
        pltpu.VMEM((H_GROUP,), jnp.int32),
        pltpu.VMEM((G, D), jnp.float32),
        pltpu.VMEM((G, D), jnp.float32),
        pltpu.VMEM((G,), jnp.float32),
        pltpu.VMEM((ACC_ROWS,), jnp.float32),
        pltpu.VMEM_SHARED((NP, D), jnp.float32),
        pltpu.VMEM_SHARED((NP,), jnp.float32),
        pltpu.SemaphoreType.DMA,
    ],
)
def _sc_pass1(srcp, dstp, typep, node_ids_p, ent, neg_rel,
              h_out, a1_out, deg_out,
              isrc, idst, ityp, idx2, ih, rows, erows, ones, degbuf,
              acc, dega, sem):
    c = lax.axis_index("c")
    s = lax.axis_index("s")
    wid = c * NS + s
    rbase = s * ACC_ROWS

    _zero_vmem_2d(rows, G, D)
    for k in range(ACC_ZERO_BLKS):
        pltpu.sync_copy(rows, acc.at[pl.ds(rbase + k * G, G)])
    _fill_vmem_1d(ones, G, 0.0)
    for k in range(ACC_ZERO_BLKS):
        pltpu.sync_copy(ones, dega.at[pl.ds(rbase + k * G, G)])
    _fill_vmem_1d(ones, G, 1.0)

    plsc.subcore_barrier()

    hbase = wid * ROWS_PER_TILE
    for g in range(ROWS_PER_TILE // H_GROUP):
        b = hbase + g * H_GROUP
        pltpu.sync_copy(node_ids_p.at[pl.ds(b, H_GROUP)], ih)
        pltpu.async_copy(ent.at[ih], rows.at[pl.ds(0, H_GROUP)], sem).wait()
        pltpu.sync_copy(rows.at[pl.ds(0, H_GROUP)], h_out.at[pl.ds(b, H_GROUP)])

    ebase = wid * EPT

    def edge_body(g, _):
        gb = ebase + g * G
        pltpu.sync_copy(srcp.at[pl.ds(gb, G)], isrc)
        pltpu.sync_copy(dstp.at[pl.ds(gb, G)], idst)
        pltpu.sync_copy(typep.at[pl.ds(gb, G)], ityp)
        pltpu.async_copy(node_ids_p.at[isrc], idx2, sem).wait()
        pltpu.async_copy(ent.at[idx2], rows, sem).wait()
        pltpu.async_copy(neg_rel.at[ityp], erows, sem).wait()
        pltpu.sync_copy(rows, acc.at[idst], add=True)
        pltpu.sync_copy(erows, acc.at[idst], add=True)
        pltpu.sync_copy(ones, dega.at[idst], add=True)
        return 0

    lax.fori_loop(0, GROUPS, edge_body, 0)

    plsc.subcore_barrier()

    pltpu.sync_copy(acc.at[pl.ds(rbase, ACC_ROWS)],
                    a1_out.at[c, pl.ds(rbase, ACC_ROWS)])
    pltpu.sync_copy(dega.at[pl.ds(rbase, ACC_ROWS)], degbuf)
    pltpu.sync_copy(degbuf, deg_out.at[pl.ds(c * NP + rbase, ACC_ROWS)])


@functools.partial(
    pl.kernel,
    out_type=[
        jax.ShapeDtypeStruct((NC, NP, OUT), jnp.float32),
    ],
    mesh=_mesh,
    scratch_types=[
        pltpu.VMEM((G,), jnp.int32),
        pltpu.VMEM((G,), jnp.int32),
        pltpu.VMEM((G,), jnp.int32),
        pltpu.VMEM((G, OUT), jnp.float32),
        pltpu.VMEM((G, OUT), jnp.float32),
        pltpu.VMEM_SHARED((NP, OUT), jnp.float32),
        pltpu.SemaphoreType.DMA,
    ],
    compiler_params=pltpu.CompilerParams(use_tc_tiling_on_sc=False),
)
def _sc_pass2(srcp, dstp, typep, h1, neg_e1,
              a2_out,
              isrc, idst, ityp, rows, erows, acc, sem):
    c = lax.axis_index("c")
    s = lax.axis_index("s")
    wid = c * NS + s
    rbase = s * ACC_ROWS

    _zero_vmem_2d(rows, G, OUT)
    for k in range(ACC_ZERO_BLKS):
        pltpu.sync_copy(rows, acc.at[pl.ds(rbase + k * G, G)])

    plsc.subcore_barrier()

    ebase = wid * EPT

    def edge_body(g, _):
        gb = ebase + g * G
        pltpu.sync_copy(srcp.at[pl.ds(gb, G)], isrc)
        pltpu.sync_copy(dstp.at[pl.ds(gb, G)], idst)
        pltpu.sync_copy(typep.at[pl.ds(gb, G)], ityp)
        pltpu.async_copy(h1.at[isrc], rows, sem).wait()
        pltpu.async_copy(neg_e1.at[ityp], erows, sem).wait()
        pltpu.sync_copy(rows, acc.at[idst], add=True)
        pltpu.sync_copy(erows, acc.at[idst], add=True)
        return 0

    lax.fori_loop(0, GROUPS, edge_body, 0)

    plsc.subcore_barrier()

    pltpu.sync_copy(acc.at[pl.ds(rbase, ACC_ROWS)],
                    a2_out.at[c, pl.ds(rbase, ACC_ROWS)])


def _tc1_body(a1p, degp, h, rel, wm1, ws1, wr1, h1_o, nege1_o, norm_o):
    a1 = a1p[0] + a1p[1]
    deg = degp[0] + degp[1]
    norm = 1.0 / jnp.maximum(deg, 1.0)
    agg = jnp.dot(a1, wm1[...], preferred_element_type=jnp.float32)
    self1 = jnp.dot(h[...], ws1[...], preferred_element_type=jnp.float32)
    h1_o[...] = jnp.maximum(agg * norm[:, None] + self1, 0.0)
    nege1_o[...] = -jnp.dot(rel[...], wr1[...], preferred_element_type=jnp.float32)
    norm_o[...] = norm


def _tc2_body(a2p, h1, normp, wm2, ws2, h2_o):
    a2 = a2p[0] + a2p[1]
    agg = jnp.dot(a2, wm2[...], preferred_element_type=jnp.float32)
    self2 = jnp.dot(h1[...], ws2[...], preferred_element_type=jnp.float32)
    h2_o[...] = jnp.maximum(agg * normp[...][:, None] + self2, 0.0)


def kernel(node_ids, edge_index, edge_type, ent_embeds, rel_embeds,
           W_msg1, W_self1, W_rel1, W_msg2, W_self2):
    i32 = jnp.int32
    src = edge_index[0].astype(i32)
    dst = edge_index[1].astype(i32)
    typ = edge_type.astype(i32)
    pad = EP - E
    srcp = jnp.concatenate([src, jnp.zeros((pad,), i32)])
    dstp = jnp.concatenate([dst, jnp.full((pad,), N, i32)])
    typep = jnp.concatenate([typ, jnp.zeros((pad,), i32)])
    node_ids_p = jnp.concatenate(
        [node_ids.astype(i32), jnp.zeros((NP - N,), i32)])
    neg_rel = -rel_embeds

    h, a1p, degp = _sc_pass1(srcp, dstp, typep, node_ids_p, ent_embeds,
                             neg_rel)
    degp = degp.reshape(NC, NP)

    h1, neg_e1, norm = pl.pallas_call(
        _tc1_body,
        out_shape=[
            jax.ShapeDtypeStruct((NP, OUT), jnp.float32),
            jax.ShapeDtypeStruct((R, OUT), jnp.float32),
            jax.ShapeDtypeStruct((NP,), jnp.float32),
        ],
    )(a1p, degp, h, rel_embeds, W_msg1, W_self1, W_rel1)

    (a2p,) = _sc_pass2(srcp, dstp, typep, h1, neg_e1)

    h2 = pl.pallas_call(
        _tc2_body,
        out_shape=jax.ShapeDtypeStruct((NP, D), jnp.float32),
    )(a2p, h1, norm, W_msg2, W_self2)

    return h2[:N]


# speedup vs baseline: 5.1007x; 1.5558x over previous
import functools

import jax
import jax.numpy as jnp
from jax import lax
from jax.experimental import pallas as pl
from jax.experimental.pallas import tpu as pltpu
from jax.experimental.pallas import tpu_sc as plsc

N = 10000
E = 320000
D = 128
OUT = 64
R = 256

NC = 2
NS = 16
NW = NC * NS

NP = 10240
ROWS_PER_TILE = NP // NW
ACC_ROWS = NP // NS
H_GROUP = 80
G = 128
GROUPS = 80
EPT = G * GROUPS
EP = EPT * NW
ACC_ZERO_BLKS = ACC_ROWS // G

_mesh = plsc.VectorSubcoreMesh(core_axis_name="c", subcore_axis_name="s")


def _zero_vmem_2d(ref, nrows, ncols):
    z = jnp.zeros((16,), jnp.float32)

    def body(i, _):
        r = i // (ncols // 16)
        c = (i % (ncols // 16)) * 16
        ref[r, pl.ds(c, 16)] = z
        return 0

    lax.fori_loop(0, nrows * (ncols // 16), body, 0)


def _fill_vmem_1d(ref, n, val):
    v = jnp.full((16,), val, jnp.float32)

    def body(i, _):
        ref[pl.ds(i * 16, 16)] = v
        return 0

    lax.fori_loop(0, n // 16, body, 0)


PHASES = 2
PGROUPS = GROUPS
TGROUPS = PHASES * PGROUPS


@functools.partial(
    pl.kernel,
    out_type=[
        jax.ShapeDtypeStruct((NC, NP, OUT), jnp.float32),
        jax.ShapeDtypeStruct((NC, NP, OUT), jnp.float32),
        jax.ShapeDtypeStruct((NP,), jnp.float32),
    ],
    mesh=_mesh,
    scratch_types=[
        pltpu.VMEM((PGROUPS, G), jnp.int32),
        pltpu.VMEM((PGROUPS, G), jnp.int32),
        pltpu.VMEM((PGROUPS, G), jnp.int32),
        pltpu.VMEM((NP,), jnp.int32),
        pltpu.VMEM((ACC_ROWS,), jnp.int32),
        pltpu.VMEM((G, OUT), jnp.float32),
        pltpu.VMEM((G, OUT), jnp.float32),
        pltpu.VMEM((G, OUT), jnp.float32),
        pltpu.VMEM((G, OUT), jnp.float32),
        pltpu.VMEM((G,), jnp.float32),
        pltpu.VMEM((ACC_ROWS,), jnp.float32),
        pltpu.VMEM_SHARED((NP, OUT), jnp.float32),
        pltpu.VMEM_SHARED((NP,), jnp.float32),
        pltpu.SemaphoreType.DMA,
        pltpu.SemaphoreType.DMA,
        pltpu.SemaphoreType.DMA,
        pltpu.SemaphoreType.DMA,
    ],
    compiler_params=pltpu.CompilerParams(
        needs_layout_passes=False, use_tc_tiling_on_sc=False),
)
def _sc_pass1(srcp, dstp, typep, node_ids_p, ent2, neg_rel2,
              h_out, a1_out, deg_out,
              dstb, typb, idx2b, nids, ih, rows0, rows1, erows0, erows1,
              ones, degbuf, acc, dega,
              semr0, semr1, seme0, seme1):
    c = lax.axis_index("c")
    s = lax.axis_index("s")
    rbase = s * ACC_ROWS
    cN = c * N
    cR = c * R
    rowbufs = (rows0, rows1)
    erowbufs = (erows0, erows1)
    semr = (semr0, semr1)
    seme = (seme0, seme1)

    _zero_vmem_2d(rows0, G, OUT)
    for k in range(ACC_ZERO_BLKS):
        pltpu.sync_copy(rows0, acc.at[pl.ds(rbase + k * G, G)])

    @pl.when(c == 0)
    def _():
        _fill_vmem_1d(ones, G, 0.0)
        for k in range(ACC_ZERO_BLKS):
            pltpu.sync_copy(ones, dega.at[pl.ds(rbase + k * G, G)])

    _fill_vmem_1d(ones, G, 1.0)
    pltpu.sync_copy(node_ids_p, nids)

    plsc.subcore_barrier()

    hbase = s * ACC_ROWS

    def hidx(i, _):
        ih[pl.ds(i * 16, 16)] = nids[pl.ds(hbase + i * 16, 16)] + cN
        return 0

    lax.fori_loop(0, ACC_ROWS // 16, hidx, 0)
    for g in range(ACC_ROWS // G):
        b = hbase + g * G
        pltpu.async_copy(ent2.at[ih.at[pl.ds(g * G, G)]], rows0, semr0).wait()
        pltpu.sync_copy(rows0, h_out.at[c, pl.ds(b, G)])

    def fire(g, b):
        pltpu.async_copy(ent2.at[idx2b.at[g]], rowbufs[b], semr[b])
        pltpu.async_copy(neg_rel2.at[typb.at[g]], erowbufs[b], seme[b])

    def wait(b):
        pltpu.make_async_copy(ent2.at[idx2b.at[0]], rowbufs[b], semr[b]).wait()
        pltpu.make_async_copy(neg_rel2.at[typb.at[0]], erowbufs[b],
                              seme[b]).wait()

    def scatter(g, b):
        pltpu.sync_copy(rowbufs[b], acc.at[dstb.at[g]], add=True)
        pltpu.sync_copy(erowbufs[b], acc.at[dstb.at[g]], add=True)

        @pl.when(c == 0)
        def _():
            pltpu.sync_copy(ones, dega.at[dstb.at[g]], add=True)

    for p in range(PHASES):
        gb = s * TGROUPS + p * PGROUPS
        pltpu.sync_copy(srcp.at[pl.ds(gb, PGROUPS)], idx2b)
        pltpu.sync_copy(dstp.at[pl.ds(gb, PGROUPS)], dstb)
        pltpu.sync_copy(typep.at[pl.ds(gb, PGROUPS)], typb)

        def resolve(i, _):
            r = i // (G // 16)
            col = (i % (G // 16)) * 16
            sv = idx2b[r, pl.ds(col, 16)]
            idx2b[r, pl.ds(col, 16)] = plsc.load_gather(nids, [sv]) + cN
            typb[r, pl.ds(col, 16)] = typb[r, pl.ds(col, 16)] + cR
            return 0

        lax.fori_loop(0, PGROUPS * (G // 16), resolve, 0)

        fire(0, 0)

        def pair(i, _):
            g0 = 2 * i
            fire(g0 + 1, 1)
            wait(0)
            scatter(g0, 0)
            fire(g0 + 2, 0)
            wait(1)
            scatter(g0 + 1, 1)
            return 0

        lax.fori_loop(0, PGROUPS // 2 - 1, pair, 0)
        g0 = PGROUPS - 2
        fire(g0 + 1, 1)
        wait(0)
        scatter(g0, 0)
        wait(1)
        scatter(g0 + 1, 1)

    plsc.subcore_barrier()

    pltpu.sync_copy(acc.at[pl.ds(rbase, ACC_ROWS)],
                    a1_out.at[c, pl.ds(rbase, ACC_ROWS)])

    @pl.when(c == 0)
    def _():
        pltpu.sync_copy(dega.at[pl.ds(rbase, ACC_ROWS)], degbuf)
        pltpu.sync_copy(degbuf, deg_out.at[pl.ds(rbase, ACC_ROWS)])


@functools.partial(
    pl.kernel,
    out_type=[
        jax.ShapeDtypeStruct((NC, NP, OUT), jnp.float32),
    ],
    mesh=_mesh,
    scratch_types=[
        pltpu.VMEM((GROUPS, G), jnp.int32),
        pltpu.VMEM((GROUPS, G), jnp.int32),
        pltpu.VMEM((GROUPS, G), jnp.int32),
        pltpu.VMEM((G, OUT), jnp.float32),
        pltpu.VMEM((G, OUT), jnp.float32),
        pltpu.VMEM((G, OUT), jnp.float32),
        pltpu.VMEM((G, OUT), jnp.float32),
        pltpu.VMEM_SHARED((NP, OUT), jnp.float32),
        pltpu.SemaphoreType.DMA,
        pltpu.SemaphoreType.DMA,
        pltpu.SemaphoreType.DMA,
        pltpu.SemaphoreType.DMA,
    ],
    compiler_params=pltpu.CompilerParams(
        needs_layout_passes=False, use_tc_tiling_on_sc=False),
)
def _sc_pass2(srcp, dstp, typep, h1, neg_e1,
              a2_out,
              srcb, dstb, typb, rows0, rows1, erows0, erows1, acc,
              semr0, semr1, seme0, seme1):
    c = lax.axis_index("c")
    s = lax.axis_index("s")
    wid = c * NS + s
    rbase = s * ACC_ROWS
    rowbufs = (rows0, rows1)
    erowbufs = (erows0, erows1)
    semr = (semr0, semr1)
    seme = (seme0, seme1)

    _zero_vmem_2d(rows0, G, OUT)
    for k in range(ACC_ZERO_BLKS):
        pltpu.sync_copy(rows0, acc.at[pl.ds(rbase + k * G, G)])

    plsc.subcore_barrier()

    gb = wid * GROUPS
    pltpu.sync_copy(srcp.at[pl.ds(gb, GROUPS)], srcb)
    pltpu.sync_copy(dstp.at[pl.ds(gb, GROUPS)], dstb)
    pltpu.sync_copy(typep.at[pl.ds(gb, GROUPS)], typb)

    def fire(g, b):
        pltpu.async_copy(h1.at[srcb.at[g]], rowbufs[b], semr[b])
        pltpu.async_copy(neg_e1.at[typb.at[g]], erowbufs[b], seme[b])

    def wait(b):
        pltpu.make_async_copy(h1.at[srcb.at[0]], rowbufs[b], semr[b]).wait()
        pltpu.make_async_copy(neg_e1.at[typb.at[0]], erowbufs[b], seme[b]).wait()

    def scatter(g, b):
        pltpu.sync_copy(rowbufs[b], acc.at[dstb.at[g]], add=True)
        pltpu.sync_copy(erowbufs[b], acc.at[dstb.at[g]], add=True)

    fire(0, 0)

    def pair(i, _):
        g0 = 2 * i
        fire(g0 + 1, 1)
        wait(0)
        scatter(g0, 0)
        fire(g0 + 2, 0)
        wait(1)
        scatter(g0 + 1, 1)
        return 0

    lax.fori_loop(0, GROUPS // 2 - 1, pair, 0)
    g0 = GROUPS - 2
    fire(g0 + 1, 1)
    wait(0)
    scatter(g0, 0)
    wait(1)
    scatter(g0 + 1, 1)

    plsc.subcore_barrier()

    pltpu.sync_copy(acc.at[pl.ds(rbase, ACC_ROWS)],
                    a2_out.at[c, pl.ds(rbase, ACC_ROWS)])


def _tc1_body(a1h, deg, hh, rel, wm1, ws1, wr1, h1_o, nege1_o, norm_o):
    f32 = jnp.float32
    norm = 1.0 / jnp.maximum(deg[...], 1.0)
    agg = (jnp.dot(a1h[0], wm1[:OUT, :], preferred_element_type=f32)
           + jnp.dot(a1h[1], wm1[OUT:, :], preferred_element_type=f32))
    self1 = (jnp.dot(hh[0], ws1[:OUT, :], preferred_element_type=f32)
             + jnp.dot(hh[1], ws1[OUT:, :], preferred_element_type=f32))
    h1_o[...] = jnp.maximum(agg * norm[:, None] + self1, 0.0)
    nege1_o[...] = -jnp.dot(rel[...], wr1[...], preferred_element_type=f32)
    norm_o[...] = norm


def _tc2_body(a2p, h1, normp, wm2, ws2, h2_o):
    a2 = a2p[0] + a2p[1]
    agg = jnp.dot(a2, wm2[...], preferred_element_type=jnp.float32)
    self2 = jnp.dot(h1[...], ws2[...], preferred_element_type=jnp.float32)
    h2_o[...] = jnp.maximum(agg * normp[...][:, None] + self2, 0.0)


def kernel(node_ids, edge_index, edge_type, ent_embeds, rel_embeds,
           W_msg1, W_self1, W_rel1, W_msg2, W_self2):
    i32 = jnp.int32
    src = edge_index[0].astype(i32)
    dst = edge_index[1].astype(i32)
    typ = edge_type.astype(i32)
    pad = EP - E
    srcp = jnp.concatenate([src, jnp.zeros((pad,), i32)]).reshape(EP // G, G)
    dstp = jnp.concatenate([dst, jnp.full((pad,), N, i32)]).reshape(EP // G, G)
    typep = jnp.concatenate([typ, jnp.zeros((pad,), i32)]).reshape(EP // G, G)
    node_ids_p = jnp.concatenate(
        [node_ids.astype(i32), jnp.zeros((NP - N,), i32)])
    ent2 = jnp.concatenate([ent_embeds[:, :OUT], ent_embeds[:, OUT:]], axis=0)
    neg_rel2 = jnp.concatenate(
        [-rel_embeds[:, :OUT], -rel_embeds[:, OUT:]], axis=0)

    hh, a1h, deg = _sc_pass1(srcp, dstp, typep, node_ids_p, ent2, neg_rel2)

    h1, neg_e1, norm = pl.pallas_call(
        _tc1_body,
        out_shape=[
            jax.ShapeDtypeStruct((NP, OUT), jnp.float32),
            jax.ShapeDtypeStruct((R, OUT), jnp.float32),
            jax.ShapeDtypeStruct((NP,), jnp.float32),
        ],
    )(a1h, deg, hh, rel_embeds, W_msg1, W_self1, W_rel1)

    (a2p,) = _sc_pass2(srcp, dstp, typep, h1, neg_e1)

    h2 = pl.pallas_call(
        _tc2_body,
        out_shape=jax.ShapeDtypeStruct((NP, D), jnp.float32),
    )(a2p, h1, norm, W_msg2, W_self2)

    return h2[:N]
